# baseline (speedup 1.0000x reference)
import functools

import jax
import jax.numpy as jnp
from jax import lax
from jax.experimental import pallas as pl
from jax.experimental.pallas import tpu as pltpu
from jax.experimental.pallas import tpu_sc as plsc

BATCH = 16384
DIM = 128
NC, NS = 2, 16
NW = NC * NS
NCHUNK = 2
CHUNK = BATCH // NCHUNK
BPW = CHUNK // NW


H = BPW // 2


def _gather_body(uid_hbm, iid_hbm, utab_hbm, itab_hbm, out_u, out_i,
                 idxu0, idxu1, idxi0, idxi1, b0, b1, b2,
                 si0, si1, si2, si3, sg0, sg1, sg2, sw0, sw1, sw2):
    wid = lax.axis_index("s") * NC + lax.axis_index("c")
    base = wid * BPW
    c0 = pltpu.async_copy(uid_hbm.at[pl.ds(base, H)], idxu0, si0)
    c1 = pltpu.async_copy(uid_hbm.at[pl.ds(base + H, H)], idxu1, si1)
    c2 = pltpu.async_copy(iid_hbm.at[pl.ds(base, H)], idxi0, si2)
    c3 = pltpu.async_copy(iid_hbm.at[pl.ds(base + H, H)], idxi1, si3)
    c0.wait()
    g0 = pltpu.async_copy(utab_hbm.at[idxu0], b0, sg0)
    c1.wait()
    g1 = pltpu.async_copy(utab_hbm.at[idxu1], b1, sg1)
    c2.wait()
    g2 = pltpu.async_copy(itab_hbm.at[idxi0], b2, sg2)
    g0.wait()
    w0 = pltpu.async_copy(b0, out_u.at[pl.ds(base, H)], sw0)
    g1.wait()
    w1 = pltpu.async_copy(b1, out_u.at[pl.ds(base + H, H)], sw1)
    c3.wait()
    w0.wait()
    g3 = pltpu.async_copy(itab_hbm.at[idxi1], b0, sg0)
    g2.wait()
    w2 = pltpu.async_copy(b2, out_i.at[pl.ds(base, H)], sw2)
    g3.wait()
    w3 = pltpu.async_copy(b0, out_i.at[pl.ds(base + H, H)], sw0)
    w1.wait()
    w2.wait()
    w3.wait()


_gather = pl.kernel(
    _gather_body,
    out_type=(
        jax.ShapeDtypeStruct((CHUNK, DIM), jnp.float32),
        jax.ShapeDtypeStruct((CHUNK, DIM), jnp.float32),
    ),
    mesh=plsc.VectorSubcoreMesh(core_axis_name="c", subcore_axis_name="s"),
    scratch_types=[
        pltpu.VMEM((H,), jnp.int32),
        pltpu.VMEM((H,), jnp.int32),
        pltpu.VMEM((H,), jnp.int32),
        pltpu.VMEM((H,), jnp.int32),
        pltpu.VMEM((H, DIM), jnp.float32),
        pltpu.VMEM((H, DIM), jnp.float32),
        pltpu.VMEM((H, DIM), jnp.float32),
    ] + [pltpu.SemaphoreType.DMA] * 10,
)


def _leaky(x):
    return jnp.where(x >= 0, x, 0.01 * x)


def _bdot(a, w):
    return jnp.dot(a.astype(jnp.bfloat16), w.astype(jnp.bfloat16),
                   preferred_element_type=jnp.float32)


def _mlp_body(u_ref, i_ref, w1a, w1b, b1, w2, b2, w3, b3, w4, b4, w5r, b5,
              out_ref):
    h = _bdot(u_ref[...], w1a[...])
    h = h + _bdot(i_ref[...], w1b[...])
    h = _leaky(h + b1[...])
    h = _leaky(jnp.dot(h, w2[...], preferred_element_type=jnp.float32) + b2[...])
    h = _leaky(jnp.dot(h, w3[...], preferred_element_type=jnp.float32) + b3[...])
    h = _leaky(jnp.dot(h, w4[...], preferred_element_type=jnp.float32) + b4[...])
    ht = h.T
    out_ref[...] = jnp.sum(ht * w5r[...], axis=0) + b5[0, 0]


def _mlp(u, i, W1a, W1b, b1, W2, b2, W3, b3, W4, b4, w5r, b5):
    TB = 2048
    grid = (CHUNK // TB,)
    full = lambda shape: pl.BlockSpec(shape, lambda g: (0,) * len(shape))
    return pl.pallas_call(
        _mlp_body,
        grid=grid,
        in_specs=[
            pl.BlockSpec((TB, DIM), lambda g: (g, 0)),
            pl.BlockSpec((TB, DIM), lambda g: (g, 0)),
            full((DIM, 64)), full((DIM, 64)), full((1, 64)),
            full((64, 64)), full((1, 64)),
            full((64, 32)), full((1, 32)),
            full((32, 16)), full((1, 16)),
            full((16, 1)), full((1, 1)),
        ],
        out_specs=pl.BlockSpec((TB,), lambda g: (g,)),
        out_shape=jax.ShapeDtypeStruct((CHUNK,), jnp.float32),
    )(u, i, W1a, W1b, b1, W2, b2, W3, b3, W4, b4, w5r, b5)


def kernel(uid, iid, user_table, item_table, W1, b1, W2, b2, W3, b3, W4, b4,
           W5, b5):
    wargs = (W1[:DIM], W1[DIM:], b1.reshape(1, 64),
             W2, b2.reshape(1, 64),
             W3, b3.reshape(1, 32),
             W4, b4.reshape(1, 16),
             W5, b5.reshape(1, 1))
    gathered = [
        _gather(uid[k * CHUNK:(k + 1) * CHUNK], iid[k * CHUNK:(k + 1) * CHUNK],
                user_table, item_table)
        for k in range(NCHUNK)
    ]
    outs = [_mlp(u, i, *wargs) for (u, i) in gathered]
    out = outs[0] if NCHUNK == 1 else jnp.concatenate(outs, axis=0)
    return out.reshape(BATCH, 1)


# speedup vs baseline: 1.1173x; 1.1173x over previous
import functools

import jax
import jax.numpy as jnp
from jax import lax
from jax.experimental import pallas as pl
from jax.experimental.pallas import tpu as pltpu
from jax.experimental.pallas import tpu_sc as plsc

BATCH = 16384
DIM = 128
NC, NS = 2, 16
NW = NC * NS
NCHUNK = 1
CHUNK = BATCH // NCHUNK
BPW = CHUNK // NW


H = BPW // 2


def _gather_body(uid_hbm, iid_hbm, utab_hbm, itab_hbm, out_u, out_i,
                 idxu0, idxu1, idxi0, idxi1, b0, b1, b2,
                 si0, si1, si2, si3, sg0, sg1, sg2, sw0, sw1, sw2):
    wid = lax.axis_index("s") * NC + lax.axis_index("c")
    base = wid * BPW
    c0 = pltpu.async_copy(uid_hbm.at[pl.ds(base, H)], idxu0, si0)
    c1 = pltpu.async_copy(uid_hbm.at[pl.ds(base + H, H)], idxu1, si1)
    c2 = pltpu.async_copy(iid_hbm.at[pl.ds(base, H)], idxi0, si2)
    c3 = pltpu.async_copy(iid_hbm.at[pl.ds(base + H, H)], idxi1, si3)
    c0.wait()
    g0 = pltpu.async_copy(utab_hbm.at[idxu0], b0, sg0)
    c1.wait()
    g1 = pltpu.async_copy(utab_hbm.at[idxu1], b1, sg1)
    c2.wait()
    g2 = pltpu.async_copy(itab_hbm.at[idxi0], b2, sg2)
    g0.wait()
    w0 = pltpu.async_copy(b0, out_u.at[pl.ds(base, H)], sw0)
    g1.wait()
    w1 = pltpu.async_copy(b1, out_u.at[pl.ds(base + H, H)], sw1)
    c3.wait()
    w0.wait()
    g3 = pltpu.async_copy(itab_hbm.at[idxi1], b0, sg0)
    g2.wait()
    w2 = pltpu.async_copy(b2, out_i.at[pl.ds(base, H)], sw2)
    g3.wait()
    w3 = pltpu.async_copy(b0, out_i.at[pl.ds(base + H, H)], sw0)
    w1.wait()
    w2.wait()
    w3.wait()


_gather = pl.kernel(
    _gather_body,
    out_type=(
        jax.ShapeDtypeStruct((CHUNK, DIM), jnp.float32),
        jax.ShapeDtypeStruct((CHUNK, DIM), jnp.float32),
    ),
    mesh=plsc.VectorSubcoreMesh(core_axis_name="c", subcore_axis_name="s"),
    scratch_types=[
        pltpu.VMEM((H,), jnp.int32),
        pltpu.VMEM((H,), jnp.int32),
        pltpu.VMEM((H,), jnp.int32),
        pltpu.VMEM((H,), jnp.int32),
        pltpu.VMEM((H, DIM), jnp.float32),
        pltpu.VMEM((H, DIM), jnp.float32),
        pltpu.VMEM((H, DIM), jnp.float32),
    ] + [pltpu.SemaphoreType.DMA] * 10,
)


def _leaky(x):
    return jnp.where(x >= 0, x, 0.01 * x)


def _bdot(a, w):
    return jnp.dot(a.astype(jnp.bfloat16), w.astype(jnp.bfloat16),
                   preferred_element_type=jnp.float32)


def _mlp_body(u_ref, i_ref, w1a, w1b, b1, w2, b2, w3, b3, w4, b4, w5r, b5,
              out_ref):
    h = _bdot(u_ref[...], w1a[...])
    h = h + _bdot(i_ref[...], w1b[...])
    h = _leaky(h + b1[...])
    h = _leaky(jnp.dot(h, w2[...], preferred_element_type=jnp.float32) + b2[...])
    h = _leaky(jnp.dot(h, w3[...], preferred_element_type=jnp.float32) + b3[...])
    h = _leaky(jnp.dot(h, w4[...], preferred_element_type=jnp.float32) + b4[...])
    ht = h.T
    out_ref[...] = jnp.sum(ht * w5r[...], axis=0) + b5[0, 0]


def _mlp(u, i, W1a, W1b, b1, W2, b2, W3, b3, W4, b4, w5r, b5):
    TB = 4096
    grid = (CHUNK // TB,)
    full = lambda shape: pl.BlockSpec(shape, lambda g: (0,) * len(shape))
    return pl.pallas_call(
        _mlp_body,
        grid=grid,
        in_specs=[
            pl.BlockSpec((TB, DIM), lambda g: (g, 0)),
            pl.BlockSpec((TB, DIM), lambda g: (g, 0)),
            full((DIM, 64)), full((DIM, 64)), full((1, 64)),
            full((64, 64)), full((1, 64)),
            full((64, 32)), full((1, 32)),
            full((32, 16)), full((1, 16)),
            full((16, 1)), full((1, 1)),
        ],
        out_specs=pl.BlockSpec((TB,), lambda g: (g,)),
        out_shape=jax.ShapeDtypeStruct((CHUNK,), jnp.float32),
    )(u, i, W1a, W1b, b1, W2, b2, W3, b3, W4, b4, w5r, b5)


def kernel(uid, iid, user_table, item_table, W1, b1, W2, b2, W3, b3, W4, b4,
           W5, b5):
    wargs = (W1[:DIM], W1[DIM:], b1.reshape(1, 64),
             W2, b2.reshape(1, 64),
             W3, b3.reshape(1, 32),
             W4, b4.reshape(1, 16),
             W5, b5.reshape(1, 1))
    gathered = [
        _gather(uid[k * CHUNK:(k + 1) * CHUNK], iid[k * CHUNK:(k + 1) * CHUNK],
                user_table, item_table)
        for k in range(NCHUNK)
    ]
    outs = [_mlp(u, i, *wargs) for (u, i) in gathered]
    out = outs[0] if NCHUNK == 1 else jnp.concatenate(outs, axis=0)
    return out.reshape(BATCH, 1)
